# baseline (speedup 1.0000x reference)
import functools

import jax
import jax.numpy as jnp
from jax import lax
from jax.experimental import pallas as pl
from jax.experimental.pallas import tpu as pltpu
from jax.experimental.pallas import tpu_sc as plsc

N = 10000
D = 128
E = 320000
NC, NS = 2, 16
NW = NC * NS
B = 128
NSTEP = 80
EPAD = NW * NSTEP * B
R = 10240
ROWS_PT = R // NS
BLK = 256


def _sc_mesh():
    return plsc.VectorSubcoreMesh(
        core_axis_name="c", subcore_axis_name="s", num_cores=NC, num_subcores=NS
    )


def _degree_body(src_all, zeros, ones_hbm, out, acc, didx, rows):
    cid = lax.axis_index("c")
    sid = lax.axis_index("s")
    wid = cid * NS + sid
    row0 = sid * ROWS_PT

    pltpu.sync_copy(ones_hbm, rows)
    for k in range(4):
        pltpu.sync_copy(zeros.at[pl.ds(row0, ROWS_PT)],
                        acc.at[pl.ds(row0, ROWS_PT)])
        plsc.subcore_barrier()
        pltpu.sync_copy(src_all.at[k, pl.ds(wid * NSTEP, NSTEP)], didx)

        def _step(j, carry):
            pltpu.sync_copy(rows, acc.at[didx.at[j]], add=True)
            return carry

        lax.fori_loop(0, NSTEP, _step, 0)
        plsc.subcore_barrier()
        pltpu.sync_copy(acc.at[pl.ds(row0, ROWS_PT)],
                        out.at[k, cid, pl.ds(row0, ROWS_PT)])
        plsc.subcore_barrier()


def _sc_degrees(src_all, zeros, ones_hbm):
    return pl.kernel(
        _degree_body,
        out_type=jax.ShapeDtypeStruct((4, NC, R, D), jnp.float32),
        mesh=_sc_mesh(),
        scratch_types=[
            pltpu.VMEM_SHARED((R, D), jnp.float32),
            pltpu.VMEM((NSTEP, B), jnp.int32),
            pltpu.VMEM((B, D), jnp.float32),
        ],
    )(src_all, zeros, ones_hbm)


NBUF = 2
IDXC = 40
S0 = NSTEP
S1 = 2 * NSTEP - S0


def _spmm_body(x0, x1, x2, x3, src_all, dst_all, zeros, out,
               acc, sidx, didx, rows, sg0, sg1, sg2, sg3):
    cid = lax.axis_index("c")
    sid = lax.axis_index("s")
    row0 = sid * ROWS_PT
    xs = [x0, x1, x2, x3]
    sg = [sg0, sg1, sg2, sg3]
    base = jnp.where(cid == 0, sid * S0, NS * S0 + sid * S1)

    def _run_chunk(r, xr, cbase):
        pltpu.sync_copy(src_all.at[r, pl.ds(cbase, IDXC)], sidx)
        pltpu.sync_copy(dst_all.at[r, pl.ds(cbase, IDXC)], didx)

        def _start(j, t):
            for q in range(2):
                pltpu.async_copy(xr.at[sidx.at[j, pl.ds(64 * q, 64)]],
                                 rows.at[t, pl.ds(64 * q, 64)], sg[2 * t + q])

        def _wait(j, t):
            for q in range(2):
                pltpu.make_async_copy(xr.at[sidx.at[j, pl.ds(64 * q, 64)]],
                                      rows.at[t, pl.ds(64 * q, 64)],
                                      sg[2 * t + q]).wait()

        for t in range(NBUF):
            _start(t, t)

        def _group(g, carry):
            for t in range(NBUF):
                j = g * NBUF + t
                _wait(j, t)
                pltpu.sync_copy(rows.at[t], acc.at[didx.at[j]], add=True)

                @pl.when(j + NBUF < IDXC)
                def _():
                    _start(j + NBUF, t)
            return carry

        lax.fori_loop(0, IDXC // NBUF, _group, 0)

    for r in range(4):
        pltpu.sync_copy(zeros.at[pl.ds(row0, ROWS_PT)],
                        acc.at[pl.ds(row0, ROWS_PT)])
        plsc.subcore_barrier()

        if S0 == S1:
            for c in range(S0 // IDXC):
                _run_chunk(r, xs[r], base + c * IDXC)
        else:
            @pl.when(cid == 0)
            def _():
                for c in range(S0 // IDXC):
                    _run_chunk(r, xs[r], base + c * IDXC)

            @pl.when(cid != 0)
            def _():
                for c in range(S1 // IDXC):
                    _run_chunk(r, xs[r], base + c * IDXC)

        plsc.subcore_barrier()
        pltpu.sync_copy(acc.at[pl.ds(row0, ROWS_PT)],
                        out.at[r, cid, pl.ds(row0, ROWS_PT)])
        plsc.subcore_barrier()


def _sc_spmm(x0, x1, x2, x3, src_all, dst_all, zeros):
    return pl.kernel(
        _spmm_body,
        out_type=jax.ShapeDtypeStruct((4, NC, R, D), jnp.float32),
        mesh=_sc_mesh(),
        scratch_types=[
            pltpu.VMEM_SHARED((R, D), jnp.float32),
            pltpu.VMEM((IDXC, B), jnp.int32),
            pltpu.VMEM((IDXC, B), jnp.int32),
            pltpu.VMEM((NBUF, B, D), jnp.float32),
            pltpu.SemaphoreType.DMA,
            pltpu.SemaphoreType.DMA,
            pltpu.SemaphoreType.DMA,
            pltpu.SemaphoreType.DMA,
        ],
    )(x0, x1, x2, x3, src_all, dst_all, zeros)


def _tc0_body(deg_ref, hd_ref, hs_ref, c_ref, x0_ref, x1_ref, x2_ref, x3_ref):
    deg = deg_ref[...]
    d = deg[:, 0] + deg[:, 1]
    c = lax.rsqrt(jnp.maximum(d, 1.0))
    c_ref[...] = c
    hd = hd_ref[...]
    hs = hs_ref[...]
    x0_ref[...] = hd * c[0]
    x1_ref[...] = hs * c[1]
    x2_ref[...] = hd * c[2]
    x3_ref[...] = hd * c[3]


def _tc0(deg, hd, hs):
    f32 = jnp.float32
    outs = pl.pallas_call(
        _tc0_body,
        grid=(R // BLK,),
        in_specs=[
            pl.BlockSpec((4, NC, BLK, D), lambda i: (0, 0, i, 0)),
            pl.BlockSpec((BLK, D), lambda i: (i, 0)),
            pl.BlockSpec((BLK, D), lambda i: (i, 0)),
        ],
        out_specs=[
            pl.BlockSpec((4, BLK, D), lambda i: (0, i, 0)),
            pl.BlockSpec((BLK, D), lambda i: (i, 0)),
            pl.BlockSpec((BLK, D), lambda i: (i, 0)),
            pl.BlockSpec((BLK, D), lambda i: (i, 0)),
            pl.BlockSpec((BLK, D), lambda i: (i, 0)),
        ],
        out_shape=[
            jax.ShapeDtypeStruct((4, R, D), f32),
            jax.ShapeDtypeStruct((R, D), f32),
            jax.ShapeDtypeStruct((R, D), f32),
            jax.ShapeDtypeStruct((R, D), f32),
            jax.ShapeDtypeStruct((R, D), f32),
        ],
    )(deg, hd, hs)
    return outs


def _tc_layer_body(p_ref, c_ref, w_ref, b_ref, *out_refs, last):
    p = p_ref[...]
    c = c_ref[...]
    w = w_ref[...]
    b = b_ref[...]
    agg0 = (p[0, 0] + p[0, 1]) * c[1]
    agg1 = (p[1, 0] + p[1, 1]) * c[0]
    agg2 = (p[2, 0] + p[2, 1]) * c[3]
    agg3 = (p[3, 0] + p[3, 1]) * c[2]
    f32 = jnp.float32
    hs_new = jnp.dot(agg0, w[0], preferred_element_type=f32) + b[0]
    hd_new = (jnp.dot(agg1, w[1], preferred_element_type=f32)
              + jnp.dot(agg2, w[2], preferred_element_type=f32)
              + jnp.dot(agg3, w[3], preferred_element_type=f32)
              + (b[1] + b[2] + b[3]))
    if last:
        hd_ref, hs_ref = out_refs
        hd_ref[...] = hd_new
        hs_ref[...] = hs_new
    else:
        hd_new = jnp.maximum(hd_new, 0.0)
        hs_new = jnp.maximum(hs_new, 0.0)
        x0_ref, x1_ref, x2_ref, x3_ref = out_refs
        x0_ref[...] = hd_new * c[0]
        x1_ref[...] = hs_new * c[1]
        x2_ref[...] = hd_new * c[2]
        x3_ref[...] = hd_new * c[3]


def _tc_layer(p, c, w, b, last):
    f32 = jnp.float32
    n_out = 2 if last else 4
    outs = pl.pallas_call(
        functools.partial(_tc_layer_body, last=last),
        grid=(R // BLK,),
        in_specs=[
            pl.BlockSpec((4, NC, BLK, D), lambda i: (0, 0, i, 0)),
            pl.BlockSpec((4, BLK, D), lambda i: (0, i, 0)),
            pl.BlockSpec((4, D, D), lambda i: (0, 0, 0)),
            pl.BlockSpec((4, 1, D), lambda i: (0, 0, 0)),
        ],
        out_specs=[pl.BlockSpec((BLK, D), lambda i: (i, 0))] * n_out,
        out_shape=[jax.ShapeDtypeStruct((R, D), f32)] * n_out,
    )(p, c, w, b)
    return outs


def kernel(relate_src, relate_dst, similar_src, similar_dst,
           embed_drug, embed_side, W, b):
    f32 = jnp.float32
    i32 = jnp.int32

    hd = jnp.zeros((R, D), f32).at[:N].set(embed_drug.astype(f32))
    hs = jnp.zeros((R, D), f32).at[:N].set(embed_side.astype(f32))

    def prep(a):
        pad = N + (jnp.arange(EPAD - E, dtype=i32) % (R - N))
        return jnp.concatenate([a.astype(i32), pad]).reshape(NW * NSTEP, B)

    rs, rd = prep(relate_src), prep(relate_dst)
    ss, sd = prep(similar_src), prep(similar_dst)
    src_all = jnp.stack([rs, rd, ss, sd])
    dst_all = jnp.stack([rd, rs, sd, ss])

    zeros_feat = jnp.zeros((R, D), f32)
    ones_hbm = jnp.ones((B, D), f32)
    bb = b.astype(f32).reshape(3, 4, 1, D)

    deg = _sc_degrees(src_all, zeros_feat, ones_hbm)
    c, x0, x1, x2, x3 = _tc0(deg, hd, hs)
    for l in range(3):
        p = _sc_spmm(x0, x1, x2, x3, src_all, dst_all, zeros_feat)
        if l < 2:
            x0, x1, x2, x3 = _tc_layer(p, c, W[l], bb[l], last=False)
        else:
            hd_out, hs_out = _tc_layer(p, c, W[l], bb[l], last=True)
    return hd_out[:N], hs_out[:N]


# speedup vs baseline: 1.0204x; 1.0204x over previous
import functools

import jax
import jax.numpy as jnp
from jax import lax
from jax.experimental import pallas as pl
from jax.experimental.pallas import tpu as pltpu
from jax.experimental.pallas import tpu_sc as plsc

N = 10000
D = 128
E = 320000
NC, NS = 2, 16
NW = NC * NS
B = 128
NSTEP = 80
EPAD = NW * NSTEP * B
R = 10240
ROWS_PT = R // NS
BLK = 256


def _sc_mesh():
    return plsc.VectorSubcoreMesh(
        core_axis_name="c", subcore_axis_name="s", num_cores=NC, num_subcores=NS
    )


def _degree_body(src_all, zeros, ones_hbm, out, acc, didx, rows):
    cid = lax.axis_index("c")
    sid = lax.axis_index("s")
    wid = cid * NS + sid
    row0 = sid * ROWS_PT

    pltpu.sync_copy(ones_hbm, rows)
    for k in range(4):
        pltpu.sync_copy(zeros.at[pl.ds(row0, ROWS_PT)],
                        acc.at[pl.ds(row0, ROWS_PT)])
        plsc.subcore_barrier()
        pltpu.sync_copy(src_all.at[k, pl.ds(wid * NSTEP, NSTEP)], didx)

        def _step(j, carry):
            pltpu.sync_copy(rows, acc.at[didx.at[j]], add=True)
            return carry

        lax.fori_loop(0, NSTEP, _step, 0)
        plsc.subcore_barrier()
        pltpu.sync_copy(acc.at[pl.ds(row0, ROWS_PT)],
                        out.at[k, cid, pl.ds(row0, ROWS_PT)])
        plsc.subcore_barrier()


def _sc_degrees(src_all, zeros, ones_hbm):
    return pl.kernel(
        _degree_body,
        out_type=jax.ShapeDtypeStruct((4, NC, R, D), jnp.float32),
        mesh=_sc_mesh(),
        scratch_types=[
            pltpu.VMEM_SHARED((R, D), jnp.float32),
            pltpu.VMEM((NSTEP, B), jnp.int32),
            pltpu.VMEM((B, D), jnp.float32),
        ],
    )(src_all, zeros, ones_hbm)


NBUF = 2
IDXC = 40
S0 = NSTEP
S1 = 2 * NSTEP - S0


def _spmm_body(x0, x1, x2, x3, src_all, dst_all, zeros, out,
               acc, sidx, didx, rows, sg0, sg1):
    cid = lax.axis_index("c")
    sid = lax.axis_index("s")
    row0 = sid * ROWS_PT
    xs = [x0, x1, x2, x3]
    sg = [sg0, sg1]
    base = jnp.where(cid == 0, sid * S0, NS * S0 + sid * S1)

    def _run_chunk(r, xr, cbase):
        pltpu.sync_copy(src_all.at[r, pl.ds(cbase, IDXC)], sidx)
        pltpu.sync_copy(dst_all.at[r, pl.ds(cbase, IDXC)], didx)
        for t in range(NBUF):
            pltpu.async_copy(xr.at[sidx.at[t]], rows.at[t], sg[t])

        def _group(g, carry):
            for t in range(NBUF):
                j = g * NBUF + t
                pltpu.make_async_copy(xr.at[sidx.at[j]],
                                      rows.at[t], sg[t]).wait()
                pltpu.sync_copy(rows.at[t], acc.at[didx.at[j]], add=True)

                @pl.when(j + NBUF < IDXC)
                def _():
                    pltpu.async_copy(xr.at[sidx.at[j + NBUF]],
                                     rows.at[t], sg[t])
            return carry

        lax.fori_loop(0, IDXC // NBUF, _group, 0)

    for r in range(4):
        pltpu.sync_copy(zeros.at[pl.ds(row0, ROWS_PT)],
                        acc.at[pl.ds(row0, ROWS_PT)])
        plsc.subcore_barrier()

        if S0 == S1:
            for c in range(S0 // IDXC):
                _run_chunk(r, xs[r], base + c * IDXC)
        else:
            @pl.when(cid == 0)
            def _():
                for c in range(S0 // IDXC):
                    _run_chunk(r, xs[r], base + c * IDXC)

            @pl.when(cid != 0)
            def _():
                for c in range(S1 // IDXC):
                    _run_chunk(r, xs[r], base + c * IDXC)

        plsc.subcore_barrier()
        pltpu.sync_copy(acc.at[pl.ds(row0, ROWS_PT)],
                        out.at[r, cid, pl.ds(row0, ROWS_PT)])
        plsc.subcore_barrier()


def _sc_spmm(x0, x1, x2, x3, src_all, dst_all, zeros):
    return pl.kernel(
        _spmm_body,
        out_type=jax.ShapeDtypeStruct((4, NC, R, D), jnp.float32),
        mesh=_sc_mesh(),
        scratch_types=[
            pltpu.VMEM_SHARED((R, D), jnp.float32),
            pltpu.VMEM((IDXC, B), jnp.int32),
            pltpu.VMEM((IDXC, B), jnp.int32),
            pltpu.VMEM((NBUF, B, D), jnp.float32),
            pltpu.SemaphoreType.DMA,
            pltpu.SemaphoreType.DMA,
        ],
    )(x0, x1, x2, x3, src_all, dst_all, zeros)


def _tc0_body(deg_ref, hd_ref, hs_ref, c_ref, x0_ref, x1_ref, x2_ref, x3_ref):
    deg = deg_ref[...]
    d = deg[:, 0] + deg[:, 1]
    c = lax.rsqrt(jnp.maximum(d, 1.0))
    c_ref[...] = c
    hd = hd_ref[...]
    hs = hs_ref[...]
    x0_ref[...] = hd * c[0]
    x1_ref[...] = hs * c[1]
    x2_ref[...] = hd * c[2]
    x3_ref[...] = hd * c[3]


def _tc0(deg, hd, hs):
    f32 = jnp.float32
    outs = pl.pallas_call(
        _tc0_body,
        grid=(R // BLK,),
        in_specs=[
            pl.BlockSpec((4, NC, BLK, D), lambda i: (0, 0, i, 0)),
            pl.BlockSpec((BLK, D), lambda i: (i, 0)),
            pl.BlockSpec((BLK, D), lambda i: (i, 0)),
        ],
        out_specs=[
            pl.BlockSpec((4, BLK, D), lambda i: (0, i, 0)),
            pl.BlockSpec((BLK, D), lambda i: (i, 0)),
            pl.BlockSpec((BLK, D), lambda i: (i, 0)),
            pl.BlockSpec((BLK, D), lambda i: (i, 0)),
            pl.BlockSpec((BLK, D), lambda i: (i, 0)),
        ],
        out_shape=[
            jax.ShapeDtypeStruct((4, R, D), f32),
            jax.ShapeDtypeStruct((R, D), f32),
            jax.ShapeDtypeStruct((R, D), f32),
            jax.ShapeDtypeStruct((R, D), f32),
            jax.ShapeDtypeStruct((R, D), f32),
        ],
    )(deg, hd, hs)
    return outs


def _tc_layer_body(p_ref, c_ref, w_ref, b_ref, *out_refs, last):
    p = p_ref[...]
    c = c_ref[...]
    w = w_ref[...]
    b = b_ref[...]
    agg0 = (p[0, 0] + p[0, 1]) * c[1]
    agg1 = (p[1, 0] + p[1, 1]) * c[0]
    agg2 = (p[2, 0] + p[2, 1]) * c[3]
    agg3 = (p[3, 0] + p[3, 1]) * c[2]
    f32 = jnp.float32
    hs_new = jnp.dot(agg0, w[0], preferred_element_type=f32) + b[0]
    hd_new = (jnp.dot(agg1, w[1], preferred_element_type=f32)
              + jnp.dot(agg2, w[2], preferred_element_type=f32)
              + jnp.dot(agg3, w[3], preferred_element_type=f32)
              + (b[1] + b[2] + b[3]))
    if last:
        hd_ref, hs_ref = out_refs
        hd_ref[...] = hd_new
        hs_ref[...] = hs_new
    else:
        hd_new = jnp.maximum(hd_new, 0.0)
        hs_new = jnp.maximum(hs_new, 0.0)
        x0_ref, x1_ref, x2_ref, x3_ref = out_refs
        x0_ref[...] = hd_new * c[0]
        x1_ref[...] = hs_new * c[1]
        x2_ref[...] = hd_new * c[2]
        x3_ref[...] = hd_new * c[3]


def _tc_layer(p, c, w, b, last):
    f32 = jnp.float32
    n_out = 2 if last else 4
    outs = pl.pallas_call(
        functools.partial(_tc_layer_body, last=last),
        grid=(R // BLK,),
        in_specs=[
            pl.BlockSpec((4, NC, BLK, D), lambda i: (0, 0, i, 0)),
            pl.BlockSpec((4, BLK, D), lambda i: (0, i, 0)),
            pl.BlockSpec((4, D, D), lambda i: (0, 0, 0)),
            pl.BlockSpec((4, 1, D), lambda i: (0, 0, 0)),
        ],
        out_specs=[pl.BlockSpec((BLK, D), lambda i: (i, 0))] * n_out,
        out_shape=[jax.ShapeDtypeStruct((R, D), f32)] * n_out,
    )(p, c, w, b)
    return outs


def kernel(relate_src, relate_dst, similar_src, similar_dst,
           embed_drug, embed_side, W, b):
    f32 = jnp.float32
    i32 = jnp.int32

    hd = jnp.zeros((R, D), f32).at[:N].set(embed_drug.astype(f32))
    hs = jnp.zeros((R, D), f32).at[:N].set(embed_side.astype(f32))

    def prep(a):
        pad = N + (jnp.arange(EPAD - E, dtype=i32) % (R - N))
        return jnp.concatenate([a.astype(i32), pad]).reshape(NW * NSTEP, B)

    rs, rd = prep(relate_src), prep(relate_dst)
    ss, sd = prep(similar_src), prep(similar_dst)
    src_all = jnp.stack([rs, rd, ss, sd])
    dst_all = jnp.stack([rd, rs, sd, ss])

    zeros_feat = jnp.zeros((R, D), f32)
    ones_hbm = jnp.ones((B, D), f32)
    bb = b.astype(f32).reshape(3, 4, 1, D)

    deg = _sc_degrees(src_all, zeros_feat, ones_hbm)
    c, x0, x1, x2, x3 = _tc0(deg, hd, hs)
    for l in range(3):
        p = _sc_spmm(x0, x1, x2, x3, src_all, dst_all, zeros_feat)
        if l < 2:
            x0, x1, x2, x3 = _tc_layer(p, c, W[l], bb[l], last=False)
        else:
            hd_out, hs_out = _tc_layer(p, c, W[l], bb[l], last=True)
    return hd_out[:N], hs_out[:N]
